# baseline (speedup 1.0000x reference)
import functools

import jax
import jax.numpy as jnp
from jax import lax
from jax.experimental import pallas as pl
from jax.experimental.pallas import tpu as pltpu
from jax.experimental.pallas import tpu_sc as plsc

B = 16384
E = 64
G = 32
H = 128

NC = 2
NS = 16
NW = NC * NS
BPW = B // NW
CHUNK = 128


def _sc_gather_one(ids, table4):
    mesh = plsc.VectorSubcoreMesh(core_axis_name="c", subcore_axis_name="s")
    L = 16
    NQ = E // L
    NCH = BPW // CHUNK

    @functools.partial(
        pl.kernel,
        mesh=mesh,
        compiler_params=pltpu.CompilerParams(
            use_tc_tiling_on_sc=False, needs_layout_passes=False),
        out_type=jax.ShapeDtypeStruct((B, E), jnp.float32),
        scratch_types=(
            [pltpu.VMEM((BPW,), jnp.int32)]
            + [pltpu.VMEM((CHUNK,), jnp.int32) for _ in range(NQ)]
            + [pltpu.VMEM((CHUNK, L), jnp.float32) for _ in range(NQ)]
            + [
                pltpu.VMEM((BPW, E), jnp.float32),
                pltpu.SemaphoreType.DMA,
                pltpu.SemaphoreType.DMA,
            ]
        ),
    )
    def gk(ids_hbm, tab_hbm, out_hbm, *scratch):
        idx = scratch[0]
        qidx = scratch[1:1 + NQ]
        qbuf = scratch[1 + NQ:1 + 2 * NQ]
        rows_v, sem, sem2 = scratch[1 + 2 * NQ:]
        wid = lax.axis_index("s") * NC + lax.axis_index("c")
        base = wid * BPW
        pltpu.sync_copy(ids_hbm.at[pl.ds(base, BPW)], idx)

        @pl.loop(0, NCH)
        def chunk(j):
            cbase = j * CHUNK
            for q in range(NQ):
                for k in range(CHUNK // L):
                    i16 = pl.ds(k * L, L)
                    v = idx[pl.ds(cbase + k * L, L)]
                    qidx[q][i16] = v * 4 + q
            gathers = [
                pltpu.async_copy(tab_hbm.at[qidx[q]], qbuf[q], sem)
                for q in range(NQ)
            ]
            for c in gathers:
                c.wait()
            for q in range(NQ):
                dst = pl.ds(q * L, L)
                for n in range(CHUNK):
                    rows_v[cbase + n, dst] = qbuf[q][n]

        pltpu.sync_copy(rows_v, out_hbm.at[pl.ds(base, BPW)])

    return gk(ids, table4)


def _sc_gather_emb(sid, aid, semb, aemb):
    se = _sc_gather_one(sid, semb.reshape(-1, 16))
    ae = _sc_gather_one(aid, aemb.reshape(-1, 16))
    return se, ae


def _sc_gather_bias(sid, aid, sbias16, abias16):
    mesh = plsc.VectorSubcoreMesh(core_axis_name="c", subcore_axis_name="s")
    L = 16
    NCH = BPW // CHUNK

    @functools.partial(
        pl.kernel,
        mesh=mesh,
        compiler_params=pltpu.CompilerParams(
            use_tc_tiling_on_sc=False, needs_layout_passes=False),
        out_type=[
            jax.ShapeDtypeStruct((B,), jnp.float32),
            jax.ShapeDtypeStruct((B,), jnp.float32),
        ],
        scratch_types=(
            [pltpu.VMEM((CHUNK,), jnp.int32)] * NCH
            + [pltpu.VMEM((CHUNK,), jnp.int32)] * NCH
            + [
                pltpu.VMEM((NCH, CHUNK), jnp.int32),
                pltpu.VMEM((NCH, CHUNK), jnp.int32),
                pltpu.VMEM((NCH, CHUNK), jnp.int32),
                pltpu.VMEM((NCH, CHUNK), jnp.int32),
                pltpu.VMEM((BPW, L), jnp.float32),
                pltpu.VMEM((BPW, L), jnp.float32),
                pltpu.VMEM((BPW,), jnp.float32),
                pltpu.VMEM((BPW,), jnp.float32),
                pltpu.SemaphoreType.DMA,
            ]
        ),
    )
    def gk(sid_hbm, aid_hbm, sbias_hbm, abias_hbm,
           sb_out, ab_out, *scratch):
        shi = scratch[0:NCH]
        ahi = scratch[NCH:2 * NCH]
        (sidx, aidx, slo, alo, sbrows, abrows, sb_v, ab_v, sem) = scratch[2 * NCH:]
        wid = lax.axis_index("s") * NC + lax.axis_index("c")
        base = wid * BPW
        for j in range(NCH):
            pltpu.sync_copy(
                sid_hbm.at[pl.ds(base + j * CHUNK, CHUNK)], sidx.at[j])
            pltpu.sync_copy(
                aid_hbm.at[pl.ds(base + j * CHUNK, CHUNK)], aidx.at[j])
        for j in range(NCH):
            for k in range(CHUNK // L):
                cols = pl.ds(k * L, L)
                sv = sidx[j, cols]
                av = aidx[j, cols]
                shi[j][cols] = lax.shift_right_logical(sv, 4)
                slo[j, cols] = lax.bitwise_and(sv, 15)
                ahi[j][cols] = lax.shift_right_logical(av, 4)
                alo[j, cols] = lax.bitwise_and(av, 15)
        copies = []
        for j in range(NCH):
            rows = pl.ds(j * CHUNK, CHUNK)
            copies.append(pltpu.async_copy(sbias_hbm.at[shi[j]], sbrows.at[rows], sem))
            copies.append(pltpu.async_copy(abias_hbm.at[ahi[j]], abrows.at[rows], sem))
        for c in copies:
            c.wait()
        for c in range(BPW // L):
            j, k = divmod(c, CHUNK // L)
            cols = pl.ds(k * L, L)
            rid = lax.iota(jnp.int32, L) + c * L
            sb_v[pl.ds(c * L, L)] = plsc.load_gather(sbrows, [rid, slo[j, cols]])
            ab_v[pl.ds(c * L, L)] = plsc.load_gather(abrows, [rid, alo[j, cols]])
        out_rows = pl.ds(base, BPW)
        pltpu.sync_copy(sb_v, sb_out.at[out_rows])
        pltpu.sync_copy(ab_v, ab_out.at[out_rows])

    return gk(sid, aid, sbias16, abias16)


def _tc_body(genreT_ref, se_ref, ae_ref, sb_ref, ab_ref, gw_ref, gb_ref,
             w1s_ref, w1a_ref, w1g_ref, b1_ref, w2_ref, cc_ref, out_ref):
    dnT = (((1,), (0,)), ((), ()))
    dnR = (((1,), (1,)), ((), ()))
    gT = jnp.maximum(
        lax.dot_general(gw_ref[...], genreT_ref[...], dnT) + gb_ref[...], 0.0)
    se = se_ref[...]
    ae = ae_ref[...]
    hT = (lax.dot_general(w1s_ref[...], se, dnR)
          + lax.dot_general(w1a_ref[...], ae, dnR)
          + lax.dot_general(w1g_ref[...], gT, dnT)
          + b1_ref[...])
    hT = jnp.maximum(hT, 0.0)
    mlpT = lax.dot_general(w2_ref[...], hT, dnT)
    ones = jnp.ones((1, E), jnp.float32)
    dotT = lax.dot_general(ones, se * ae, dnR)
    out_ref[...] = (dotT + mlpT + sb_ref[0] + ab_ref[0] + cc_ref[...])[None]


def _tc_forward(genreT, se, ae, sb3, ab3, gW, gbc, w1s, w1a, w1g, b1c, w2, cc):
    NGEN = genreT.shape[0]
    BB = 512
    grid = (B // BB,)
    return pl.pallas_call(
        _tc_body,
        grid=grid,
        in_specs=[
            pl.BlockSpec((NGEN, BB), lambda i: (0, i)),
            pl.BlockSpec((BB, E), lambda i: (i, 0)),
            pl.BlockSpec((BB, E), lambda i: (i, 0)),
            pl.BlockSpec((1, 1, BB), lambda i: (i, 0, 0)),
            pl.BlockSpec((1, 1, BB), lambda i: (i, 0, 0)),
            pl.BlockSpec((G, NGEN), lambda i: (0, 0)),
            pl.BlockSpec((G, 1), lambda i: (0, 0)),
            pl.BlockSpec((H, E), lambda i: (0, 0)),
            pl.BlockSpec((H, E), lambda i: (0, 0)),
            pl.BlockSpec((H, G), lambda i: (0, 0)),
            pl.BlockSpec((H, 1), lambda i: (0, 0)),
            pl.BlockSpec((1, H), lambda i: (0, 0)),
            pl.BlockSpec((1, 1), lambda i: (0, 0)),
        ],
        out_specs=pl.BlockSpec((1, 1, BB), lambda i: (i, 0, 0)),
        out_shape=jax.ShapeDtypeStruct((B // BB, 1, BB), jnp.float32),
    )(genreT, se, ae, sb3, ab3, gW, gbc, w1s, w1a, w1g, b1c, w2, cc)


def kernel(songIDs, artistIDs, genreMH, songEmb, artistEmb, songBiasT,
           artistBiasT, bias, gW, gb, w1, b1, w2, b2):
    sid = songIDs.astype(jnp.int32)
    aid = artistIDs.astype(jnp.int32)
    se, ae = _sc_gather_emb(sid, aid, songEmb, artistEmb)
    sb, ab = _sc_gather_bias(sid, aid, songBiasT.reshape(-1, 16),
                             artistBiasT.reshape(-1, 16))
    sb3 = sb.reshape(B // BPW, 1, BPW)
    ab3 = ab.reshape(B // BPW, 1, BPW)
    w1s = w1[:, :E]
    w1a = w1[:, E:2 * E]
    w1g = w1[:, 2 * E:]
    gbc = gb.reshape(G, 1)
    b1c = b1.reshape(H, 1)
    cc = (b2 + bias).reshape(1, 1)
    out3 = _tc_forward(genreMH.T, se, ae, sb3, ab3, gW, gbc, w1s, w1a, w1g,
                       b1c, w2, cc)
    return out3.reshape(B)


# speedup vs baseline: 1.0269x; 1.0269x over previous
import functools

import jax
import jax.numpy as jnp
from jax import lax
from jax.experimental import pallas as pl
from jax.experimental.pallas import tpu as pltpu
from jax.experimental.pallas import tpu_sc as plsc

B = 16384
E = 64
G = 32
H = 128

NC = 2
NS = 16
NW = NC * NS
BPW = B // NW
CHUNK = 128


def _sc_gather(sid, aid, songEmb, artistEmb, sbias16, abias16):
    mesh = plsc.VectorSubcoreMesh(core_axis_name="c", subcore_axis_name="s")
    L = 16
    NCH = BPW // CHUNK

    @functools.partial(
        pl.kernel,
        mesh=mesh,
        compiler_params=pltpu.CompilerParams(
            use_tc_tiling_on_sc=False, needs_layout_passes=False),
        out_type=[
            jax.ShapeDtypeStruct((B, E), jnp.float32),
            jax.ShapeDtypeStruct((B, E), jnp.float32),
            jax.ShapeDtypeStruct((B,), jnp.float32),
            jax.ShapeDtypeStruct((B,), jnp.float32),
        ],
        scratch_types=(
            [pltpu.VMEM((CHUNK,), jnp.int32)] * NCH
            + [pltpu.VMEM((CHUNK,), jnp.int32)] * NCH
            + [pltpu.VMEM((CHUNK,), jnp.int32)] * NCH
            + [pltpu.VMEM((CHUNK,), jnp.int32)] * NCH
            + [
                pltpu.VMEM((NCH, CHUNK), jnp.int32),
                pltpu.VMEM((NCH, CHUNK), jnp.int32),
                pltpu.VMEM((BPW, E), jnp.float32),
                pltpu.VMEM((BPW, E), jnp.float32),
                pltpu.VMEM((BPW, L), jnp.float32),
                pltpu.VMEM((BPW, L), jnp.float32),
                pltpu.VMEM((BPW,), jnp.float32),
                pltpu.VMEM((BPW,), jnp.float32),
                pltpu.SemaphoreType.DMA,
            ]
        ),
    )
    def gk(sid_hbm, aid_hbm, semb_hbm, aemb_hbm, sbias_hbm, abias_hbm,
           se_out, ae_out, sb_out, ab_out, *scratch):
        sidx = scratch[0:NCH]
        aidx = scratch[NCH:2 * NCH]
        shi = scratch[2 * NCH:3 * NCH]
        ahi = scratch[3 * NCH:4 * NCH]
        (slo, alo, se_v, ae_v, sbrows, abrows, sb_v, ab_v, sem) = scratch[4 * NCH:]
        wid = lax.axis_index("s") * NC + lax.axis_index("c")
        base = wid * BPW
        for j in range(NCH):
            pltpu.sync_copy(sid_hbm.at[pl.ds(base + j * CHUNK, CHUNK)], sidx[j])
            pltpu.sync_copy(aid_hbm.at[pl.ds(base + j * CHUNK, CHUNK)], aidx[j])
        for j in range(NCH):
            for k in range(CHUNK // L):
                cols = pl.ds(k * L, L)
                sv = sidx[j][cols]
                av = aidx[j][cols]
                shi[j][cols] = lax.shift_right_logical(sv, 4)
                slo[j, cols] = lax.bitwise_and(sv, 15)
                ahi[j][cols] = lax.shift_right_logical(av, 4)
                alo[j, cols] = lax.bitwise_and(av, 15)
        copies = []
        for j in range(NCH):
            rows = pl.ds(j * CHUNK, CHUNK)
            copies.append(pltpu.async_copy(semb_hbm.at[sidx[j]], se_v.at[rows], sem))
            copies.append(pltpu.async_copy(aemb_hbm.at[aidx[j]], ae_v.at[rows], sem))
            copies.append(pltpu.async_copy(sbias_hbm.at[shi[j]], sbrows.at[rows], sem))
            copies.append(pltpu.async_copy(abias_hbm.at[ahi[j]], abrows.at[rows], sem))
        for c in copies:
            c.wait()
        for c in range(BPW // L):
            j, k = divmod(c, CHUNK // L)
            cols = pl.ds(k * L, L)
            rid = lax.iota(jnp.int32, L) + c * L
            sb_v[pl.ds(c * L, L)] = plsc.load_gather(sbrows, [rid, slo[j, cols]])
            ab_v[pl.ds(c * L, L)] = plsc.load_gather(abrows, [rid, alo[j, cols]])
        out_rows = pl.ds(base, BPW)
        pltpu.sync_copy(se_v, se_out.at[out_rows])
        pltpu.sync_copy(ae_v, ae_out.at[out_rows])
        pltpu.sync_copy(sb_v, sb_out.at[out_rows])
        pltpu.sync_copy(ab_v, ab_out.at[out_rows])

    return gk(sid, aid, songEmb, artistEmb, sbias16, abias16)


def _tc_genre_body(genreT_ref, gw_ref, gb_ref, w1g_ref, pg_ref):
    dnT = (((1,), (0,)), ((), ()))
    gT = jnp.maximum(
        lax.dot_general(gw_ref[...], genreT_ref[...], dnT) + gb_ref[...], 0.0)
    pg_ref[...] = lax.dot_general(w1g_ref[...], gT, dnT)


def _tc_genre(genreT, gW, gbc, w1g):
    NGEN = genreT.shape[0]
    BB = 512
    return pl.pallas_call(
        _tc_genre_body,
        grid=(B // BB,),
        in_specs=[
            pl.BlockSpec((NGEN, BB), lambda i: (0, i)),
            pl.BlockSpec((G, NGEN), lambda i: (0, 0)),
            pl.BlockSpec((G, 1), lambda i: (0, 0)),
            pl.BlockSpec((H, G), lambda i: (0, 0)),
        ],
        out_specs=pl.BlockSpec((H, BB), lambda i: (0, i)),
        out_shape=jax.ShapeDtypeStruct((H, B), jnp.float32),
    )(genreT, gW, gbc, w1g)


def _tc_combine_body(pg_ref, se_ref, ae_ref, sb_ref, ab_ref,
                     w1s_ref, w1a_ref, b1_ref, w2_ref, cc_ref, out_ref):
    dnT = (((1,), (0,)), ((), ()))
    dnR = (((1,), (1,)), ((), ()))
    se = se_ref[...]
    ae = ae_ref[...]
    hT = (lax.dot_general(w1s_ref[...], se, dnR)
          + lax.dot_general(w1a_ref[...], ae, dnR)
          + pg_ref[...]
          + b1_ref[...])
    hT = jnp.maximum(hT, 0.0)
    mlpT = lax.dot_general(w2_ref[...], hT, dnT)
    ones = jnp.ones((1, E), jnp.float32)
    dotT = lax.dot_general(ones, se * ae, dnR)
    out_ref[...] = (dotT + mlpT + sb_ref[0] + ab_ref[0] + cc_ref[...])[None]


def _tc_combine(pgT, se, ae, sb3, ab3, w1s, w1a, b1c, w2, cc):
    BB = 512
    return pl.pallas_call(
        _tc_combine_body,
        grid=(B // BB,),
        in_specs=[
            pl.BlockSpec((H, BB), lambda i: (0, i)),
            pl.BlockSpec((BB, E), lambda i: (i, 0)),
            pl.BlockSpec((BB, E), lambda i: (i, 0)),
            pl.BlockSpec((1, 1, BB), lambda i: (i, 0, 0)),
            pl.BlockSpec((1, 1, BB), lambda i: (i, 0, 0)),
            pl.BlockSpec((H, E), lambda i: (0, 0)),
            pl.BlockSpec((H, E), lambda i: (0, 0)),
            pl.BlockSpec((H, 1), lambda i: (0, 0)),
            pl.BlockSpec((1, H), lambda i: (0, 0)),
            pl.BlockSpec((1, 1), lambda i: (0, 0)),
        ],
        out_specs=pl.BlockSpec((1, 1, BB), lambda i: (i, 0, 0)),
        out_shape=jax.ShapeDtypeStruct((B // BB, 1, BB), jnp.float32),
    )(pgT, se, ae, sb3, ab3, w1s, w1a, b1c, w2, cc)


def kernel(songIDs, artistIDs, genreMH, songEmb, artistEmb, songBiasT,
           artistBiasT, bias, gW, gb, w1, b1, w2, b2):
    sid = songIDs.astype(jnp.int32)
    aid = artistIDs.astype(jnp.int32)
    w1s = w1[:, :E]
    w1a = w1[:, E:2 * E]
    w1g = w1[:, 2 * E:]
    gbc = gb.reshape(G, 1)
    b1c = b1.reshape(H, 1)
    cc = (b2 + bias).reshape(1, 1)
    pgT = _tc_genre(genreMH.T, gW, gbc, w1g)
    se, ae, sb, ab = _sc_gather(sid, aid, songEmb, artistEmb,
                                songBiasT.reshape(-1, 16),
                                artistBiasT.reshape(-1, 16))
    sb3 = sb.reshape(B // BPW, 1, BPW)
    ab3 = ab.reshape(B // BPW, 1, BPW)
    out3 = _tc_combine(pgT, se, ae, sb3, ab3, w1s, w1a, b1c, w2, cc)
    return out3.reshape(B)


# speedup vs baseline: 1.0544x; 1.0268x over previous
import functools

import jax
import jax.numpy as jnp
from jax import lax
from jax.experimental import pallas as pl
from jax.experimental.pallas import tpu as pltpu
from jax.experimental.pallas import tpu_sc as plsc

B = 16384
E = 64
G = 32
H = 128

NC = 2
NS = 16
NW = NC * NS
BPW = B // NW
CHUNK = 128


def _sc_gather(sid, aid, songEmb, artistEmb, sbias16, abias16):
    mesh = plsc.VectorSubcoreMesh(core_axis_name="c", subcore_axis_name="s")
    L = 16
    NCH = BPW // CHUNK

    @functools.partial(
        pl.kernel,
        mesh=mesh,
        compiler_params=pltpu.CompilerParams(
            use_tc_tiling_on_sc=False, needs_layout_passes=False),
        out_type=[
            jax.ShapeDtypeStruct((B, E), jnp.float32),
            jax.ShapeDtypeStruct((B, E), jnp.float32),
            jax.ShapeDtypeStruct((B,), jnp.float32),
            jax.ShapeDtypeStruct((B,), jnp.float32),
        ],
        scratch_types=(
            [pltpu.VMEM((CHUNK,), jnp.int32)] * NCH
            + [pltpu.VMEM((CHUNK,), jnp.int32)] * NCH
            + [pltpu.VMEM((CHUNK,), jnp.int32)] * NCH
            + [pltpu.VMEM((CHUNK,), jnp.int32)] * NCH
            + [
                pltpu.VMEM((NCH, CHUNK), jnp.int32),
                pltpu.VMEM((NCH, CHUNK), jnp.int32),
                pltpu.VMEM((BPW, E), jnp.float32),
                pltpu.VMEM((BPW, E), jnp.float32),
                pltpu.VMEM((BPW, L), jnp.float32),
                pltpu.VMEM((BPW, L), jnp.float32),
                pltpu.VMEM((BPW,), jnp.float32),
                pltpu.VMEM((BPW,), jnp.float32),
                pltpu.SemaphoreType.DMA,
            ]
        ),
    )
    def gk(sid_hbm, aid_hbm, semb_hbm, aemb_hbm, sbias_hbm, abias_hbm,
           se_out, ae_out, sb_out, ab_out, *scratch):
        sidx = scratch[0:NCH]
        aidx = scratch[NCH:2 * NCH]
        shi = scratch[2 * NCH:3 * NCH]
        ahi = scratch[3 * NCH:4 * NCH]
        (slo, alo, se_v, ae_v, sbrows, abrows, sb_v, ab_v, sem) = scratch[4 * NCH:]
        wid = lax.axis_index("s") * NC + lax.axis_index("c")
        base = wid * BPW
        for j in range(NCH):
            pltpu.sync_copy(sid_hbm.at[pl.ds(base + j * CHUNK, CHUNK)], sidx[j])
            pltpu.sync_copy(aid_hbm.at[pl.ds(base + j * CHUNK, CHUNK)], aidx[j])
        for j in range(NCH):
            for k in range(CHUNK // L):
                cols = pl.ds(k * L, L)
                sv = sidx[j][cols]
                av = aidx[j][cols]
                shi[j][cols] = lax.shift_right_logical(sv, 4)
                slo[j, cols] = lax.bitwise_and(sv, 15)
                ahi[j][cols] = lax.shift_right_logical(av, 4)
                alo[j, cols] = lax.bitwise_and(av, 15)
        copies = []
        for j in range(NCH):
            rows = pl.ds(j * CHUNK, CHUNK)
            copies.append(pltpu.async_copy(semb_hbm.at[sidx[j]], se_v.at[rows], sem))
            copies.append(pltpu.async_copy(aemb_hbm.at[aidx[j]], ae_v.at[rows], sem))
            copies.append(pltpu.async_copy(sbias_hbm.at[shi[j]], sbrows.at[rows], sem))
            copies.append(pltpu.async_copy(abias_hbm.at[ahi[j]], abrows.at[rows], sem))
        for c in copies:
            c.wait()
        for c in range(BPW // L):
            j, k = divmod(c, CHUNK // L)
            cols = pl.ds(k * L, L)
            rid = lax.iota(jnp.int32, L) + c * L
            sb_v[pl.ds(c * L, L)] = plsc.load_gather(sbrows, [rid, slo[j, cols]])
            ab_v[pl.ds(c * L, L)] = plsc.load_gather(abrows, [rid, alo[j, cols]])
        out_rows = pl.ds(base, BPW)
        pltpu.sync_copy(se_v, se_out.at[out_rows])
        pltpu.sync_copy(ae_v, ae_out.at[out_rows])
        pltpu.sync_copy(sb_v, sb_out.at[out_rows])
        pltpu.sync_copy(ab_v, ab_out.at[out_rows])

    return gk(sid, aid, songEmb, artistEmb, sbias16, abias16)


def _tc_body(genreT_ref, se_ref, ae_ref, sb_ref, ab_ref, gw_ref, gb_ref,
             w1s_ref, w1a_ref, w1g_ref, b1_ref, w2_ref, cc_ref, out_ref):
    dnT = (((1,), (0,)), ((), ()))
    dnR = (((1,), (1,)), ((), ()))
    gT = jnp.maximum(
        lax.dot_general(gw_ref[...], genreT_ref[...], dnT) + gb_ref[...], 0.0)
    se = se_ref[...]
    ae = ae_ref[...]
    hT = (lax.dot_general(w1s_ref[...], se, dnR)
          + lax.dot_general(w1a_ref[...], ae, dnR)
          + lax.dot_general(w1g_ref[...], gT, dnT)
          + b1_ref[...])
    hT = jnp.maximum(hT, 0.0)
    mlpT = lax.dot_general(w2_ref[...], hT, dnT)
    ones = jnp.ones((1, E), jnp.float32)
    dotT = lax.dot_general(ones, se * ae, dnR)
    out_ref[...] = (dotT + mlpT + sb_ref[0] + ab_ref[0] + cc_ref[...])[None]


def _tc_forward(genreT, se, ae, sb3, ab3, gW, gbc, w1s, w1a, w1g, b1c, w2, cc):
    NGEN = genreT.shape[0]
    BB = 512
    grid = (B // BB,)
    return pl.pallas_call(
        _tc_body,
        grid=grid,
        in_specs=[
            pl.BlockSpec((NGEN, BB), lambda i: (0, i)),
            pl.BlockSpec((BB, E), lambda i: (i, 0)),
            pl.BlockSpec((BB, E), lambda i: (i, 0)),
            pl.BlockSpec((1, 1, BB), lambda i: (i, 0, 0)),
            pl.BlockSpec((1, 1, BB), lambda i: (i, 0, 0)),
            pl.BlockSpec((G, NGEN), lambda i: (0, 0)),
            pl.BlockSpec((G, 1), lambda i: (0, 0)),
            pl.BlockSpec((H, E), lambda i: (0, 0)),
            pl.BlockSpec((H, E), lambda i: (0, 0)),
            pl.BlockSpec((H, G), lambda i: (0, 0)),
            pl.BlockSpec((H, 1), lambda i: (0, 0)),
            pl.BlockSpec((1, H), lambda i: (0, 0)),
            pl.BlockSpec((1, 1), lambda i: (0, 0)),
        ],
        out_specs=pl.BlockSpec((1, 1, BB), lambda i: (i, 0, 0)),
        out_shape=jax.ShapeDtypeStruct((B // BB, 1, BB), jnp.float32),
    )(genreT, se, ae, sb3, ab3, gW, gbc, w1s, w1a, w1g, b1c, w2, cc)


def kernel(songIDs, artistIDs, genreMH, songEmb, artistEmb, songBiasT,
           artistBiasT, bias, gW, gb, w1, b1, w2, b2):
    sid = songIDs.astype(jnp.int32)
    aid = artistIDs.astype(jnp.int32)
    se, ae, sb, ab = _sc_gather(sid, aid, songEmb, artistEmb,
                                songBiasT.reshape(-1, 16),
                                artistBiasT.reshape(-1, 16))
    sb3 = sb.reshape(B // BPW, 1, BPW)
    ab3 = ab.reshape(B // BPW, 1, BPW)
    w1s = w1[:, :E]
    w1a = w1[:, E:2 * E]
    w1g = w1[:, 2 * E:]
    gbc = gb.reshape(G, 1)
    b1c = b1.reshape(H, 1)
    cc = (b2 + bias).reshape(1, 1)
    out3 = _tc_forward(genreMH.T, se, ae, sb3, ab3, gW, gbc, w1s, w1a, w1g,
                       b1c, w2, cc)
    return out3.reshape(B)
